# baseline (speedup 1.0000x reference)
import functools

import numpy as np

import jax
import jax.numpy as jnp
from jax import lax
from jax.experimental import pallas as pl
from jax.experimental.pallas import tpu as pltpu
from jax.experimental.pallas import tpu_sc as plsc

N = 4096
M = 4096
NNZ = 167772
NW = 32
KEYS_PER_W = (N * M) // NW

CHUNK = 4096
GROUPS = CHUNK // 16
SROWS = CHUNK // 128
KC_LEN = CHUNK + 32
PAD_LEN = ((NNZ + 2 * CHUNK + KC_LEN) // 8 + 1) * 8

ZWORDS = 32768
ZITER = KEYS_PER_W // ZWORDS

SENTINEL = np.int32(0x7FFFFFFF)


def _sc_body(skey_hbm, sval_hbm, starts_hbm, tvals_hbm, out_hbm,
             zbuf, kc0, kc1, vc0, vc1, pk0, pk1, pv0, pv1, sb, tb,
             zsem, isem, ssem):
    wid = lax.axis_index("s") * 2 + lax.axis_index("c")

    pltpu.sync_copy(starts_hbm, sb)
    pltpu.sync_copy(tvals_hbm, tb)
    bv = sb[pl.ds(wid, 16)]
    lo = bv[0]
    hi = bv[1]
    tval = tb[pl.ds(wid, 16)][0]
    tvsplat = jnp.full((16,), tval, jnp.float32)

    lane = lax.iota(jnp.int32, 16)
    zf16 = (lane * 0).astype(jnp.float32)

    def _zstore(i, _):
        zbuf[pl.ds(i * 16, 16)] = zf16
        return 0
    lax.fori_loop(0, ZWORDS // 16, _zstore, 0)
    slab = wid * np.int32(KEYS_PER_W)

    def _zfire(j, _):
        zoff = pl.multiple_of(slab + j * np.int32(ZWORDS), 8)
        pltpu.async_copy(zbuf, out_hbm.at[pl.ds(zoff, ZWORDS)], zsem)
        return 0
    lax.fori_loop(0, ZITER, _zfire, 0)

    def _zdrain():
        def _zwait(j, _):
            pltpu.make_async_copy(
                zbuf, out_hbm.at[pl.ds(0, ZWORDS)], zsem).wait()
            return 0
        lax.fori_loop(0, ZITER, _zwait, 0)

    lo_al = lo & np.int32(-8)
    nch = (hi - lo_al + np.int32(CHUNK - 1)) // np.int32(CHUNK)
    tsplat = jnp.full((16,), slab, jnp.int32)

    def _fire_inputs(i, kcb, vcb):
        base = pl.multiple_of(lo_al + i * np.int32(CHUNK), 8)
        pltpu.async_copy(skey_hbm.at[pl.ds(base, KC_LEN)], kcb, isem)
        pltpu.async_copy(sval_hbm.at[pl.ds(base, CHUNK)], vcb, isem)

    def _wait_inputs(kcb, vcb):
        pltpu.make_async_copy(
            skey_hbm.at[pl.ds(0, KC_LEN)], kcb, isem).wait()
        pltpu.make_async_copy(
            sval_hbm.at[pl.ds(0, CHUNK)], vcb, isem).wait()

    def _wait_scatter(pkb, pvb):
        pltpu.make_async_copy(pvb, out_hbm.at[pkb], ssem).wait()

    _fire_inputs(np.int32(0), kc0, vc0)
    _fire_inputs(np.int32(1), kc1, vc1)

    bufs = ((kc0, vc0, pk0, pv0), (kc1, vc1, pk1, pv1))

    def _process(i, kcb, vcb, pkb, pvb):
        @pl.when(i < nch)
        def _():
            @pl.when(i >= 2)
            def _():
                _wait_scatter(pkb, pvb)
            _wait_inputs(kcb, vcb)
            base = lo_al + i * np.int32(CHUNK)

            def _group(k, _):
                off = k * 16
                ka = kcb[pl.ds(off, 16)]
                kb = kcb[pl.ds(off + 1, 16)]
                va = vcb[pl.ds(off, 16)]
                gidx = base + off + lane
                keep = (ka != kb) & (gidx >= lo) & (gidx < hi)
                outk = jnp.where(keep, ka, tsplat)
                outv = jnp.where(keep, va, tvsplat)
                pkb[pl.ds(off, 16)] = outk
                pvb[pl.ds(off, 16)] = outv
                return 0
            lax.fori_loop(0, GROUPS, _group, 0)

            _fire_inputs(i + 2, kcb, vcb)

            @pl.when(i == 0)
            def _():
                _zdrain()
            pltpu.async_copy(pvb, out_hbm.at[pkb], ssem)

    def _pair(p, _):
        for b in range(2):
            kcb, vcb, pkb, pvb = bufs[b]
            _process(p * 2 + np.int32(b), kcb, vcb, pkb, pvb)
        return 0
    lax.fori_loop(0, (nch + np.int32(1)) // 2, _pair, 0)

    @pl.when(nch == 0)
    def _():
        _zdrain()
    _wait_inputs(kc0, vc0)
    _wait_inputs(kc1, vc1)

    @pl.when(nch >= 1)
    def _():
        _wait_scatter(pk0, pv0)

    @pl.when(nch >= 2)
    def _():
        _wait_scatter(pk1, pv1)


@jax.jit
def _sc_scatter(skey_pad, sval_pad, starts, tvals):
    mesh = plsc.VectorSubcoreMesh(core_axis_name="c", subcore_axis_name="s")
    f = functools.partial(
        pl.kernel,
        mesh=mesh,
        out_type=jax.ShapeDtypeStruct((N * M,), jnp.float32),
        scratch_types=[
            pltpu.VMEM((ZWORDS,), jnp.float32),
            pltpu.VMEM((KC_LEN,), jnp.int32),
            pltpu.VMEM((KC_LEN,), jnp.int32),
            pltpu.VMEM((CHUNK,), jnp.float32),
            pltpu.VMEM((CHUNK,), jnp.float32),
            pltpu.VMEM((CHUNK,), jnp.int32),
            pltpu.VMEM((CHUNK,), jnp.int32),
            pltpu.VMEM((CHUNK,), jnp.float32),
            pltpu.VMEM((CHUNK,), jnp.float32),
            pltpu.VMEM((48,), jnp.int32),
            pltpu.VMEM((48,), jnp.float32),
            pltpu.SemaphoreType.DMA,
            pltpu.SemaphoreType.DMA,
            pltpu.SemaphoreType.DMA,
        ],
    )(_sc_body)
    return f(skey_pad, sval_pad, starts, tvals)


def kernel(node_features, data, indices):
    flat = indices[:, 0] * np.int32(M) + indices[:, 1]
    skey, sval = lax.sort_key_val(flat, data, is_stable=False)

    skey_pad = jnp.full((PAD_LEN,), SENTINEL, jnp.int32).at[:NNZ].set(skey)
    sval_pad = jnp.zeros((PAD_LEN,), jnp.float32).at[:NNZ].set(sval)

    targets = jnp.arange(NW, dtype=jnp.int32) * np.int32(KEYS_PER_W)
    bounds = jnp.searchsorted(skey, targets, side="left").astype(jnp.int32)
    starts = jnp.zeros((48,), jnp.int32)
    starts = starts.at[:NW].set(bounds).at[NW].set(np.int32(NNZ))

    pr = jnp.searchsorted(skey, targets, side="right").astype(jnp.int32) - 1
    prc = jnp.maximum(pr, 0)
    exists = (pr >= 0) & (skey[prc] == targets)
    tvals = jnp.zeros((48,), jnp.float32).at[:NW].set(
        jnp.where(exists, sval[prc], 0.0))

    out = _sc_scatter(skey_pad, sval_pad, starts, tvals)
    return out.reshape(N, M)


# speedup vs baseline: 3.3203x; 3.3203x over previous
import functools

import numpy as np

import jax
import jax.numpy as jnp
from jax import lax
from jax.experimental import pallas as pl
from jax.experimental.pallas import tpu as pltpu
from jax.experimental.pallas import tpu_sc as plsc

N = 4096
M = 4096
NNZ = 167772
NW = 32
P = 10
NREG = NW * P
W = 52432
WLAST = N * M - (NREG - 1) * W
ZI = W // 16

CH = 2048
GR = CH // 16
KCH = CH + 32
PAD_LEN = ((NNZ + 2 * CH + KCH) // 8 + 1) * 8
SB_LEN = 352

SENTINEL = np.int32(0x7FFFFFFF)


def _sc_body(skey_hbm, sval_hbm, starts_hbm, tvals_hbm, out_hbm,
             win0, win1, kc0, kc1, vc0, vc1, si0, si1, sb, tb,
             isem, osem):
    wid = lax.axis_index("s") * 2 + lax.axis_index("c")
    r0 = wid * np.int32(P)

    pltpu.sync_copy(starts_hbm, sb)
    pltpu.sync_copy(tvals_hbm, tb)

    lane = lax.iota(jnp.int32, 16)
    zf16 = (lane * 0).astype(jnp.float32)

    def _region_info(p):
        bv = sb[pl.ds(r0 + p, 16)]
        lo = bv[0]
        hi = bv[1]
        lo_al = lo & np.int32(-8)
        return lo, hi, lo_al

    def _fire_inputs(p, kcb, vcb):
        _, _, lo_al = _region_info(p)
        abase = pl.multiple_of(lo_al, 8)
        pltpu.async_copy(skey_hbm.at[pl.ds(abase, KCH)], kcb, isem)
        pltpu.async_copy(sval_hbm.at[pl.ds(abase, CH)], vcb, isem)

    def _wait_inputs(kcb, vcb):
        pltpu.make_async_copy(
            skey_hbm.at[pl.ds(0, KCH)], kcb, isem).wait()
        pltpu.make_async_copy(
            sval_hbm.at[pl.ds(0, CH)], vcb, isem).wait()

    def _wait_out_w(winb):
        pltpu.make_async_copy(
            winb, out_hbm.at[pl.ds(0, W)], osem).wait()

    def _wait_out_last(winb):
        pltpu.make_async_copy(
            winb.at[pl.ds(0, WLAST)], out_hbm.at[pl.ds(0, WLAST)],
            osem).wait()

    def _zero_full(winb):
        def _z(i, _):
            winb[pl.ds(i * 16, 16)] = zf16
            return 0
        lax.fori_loop(0, ZI, _z, 0)

    _zero_full(win0)
    _zero_full(win1)
    _fire_inputs(np.int32(0), kc0, vc0)
    _fire_inputs(np.int32(1), kc1, vc1)

    bufs = ((win0, kc0, vc0, si0), (win1, kc1, vc1, si1))

    def _process(p, winb, kcb, vcb, sib):
        r = r0 + p
        lo, hi, lo_al = _region_info(p)
        nch = (hi - lo_al + np.int32(CH - 1)) // np.int32(CH)
        wstart = r * np.int32(W)
        wsplat = jnp.full((16,), wstart, jnp.int32)
        tval = tb[pl.ds(r, 16)][0]
        tvsplat = jnp.full((16,), tval, jnp.float32)

        @pl.when(p >= 2)
        def _():
            _wait_out_w(winb)
            plo, phi, plo_al = _region_info(p - 2)
            pnch = (phi - plo_al + np.int32(CH - 1)) // np.int32(CH)

            @pl.when(pnch <= 1)
            def _():
                png = (phi - plo_al + np.int32(15)) // np.int32(16)

                def _rz(k, _):
                    idxv = sib[pl.ds(k * 16, 16)]
                    plsc.store_scatter(winb, [idxv], zf16)
                    return 0
                lax.fori_loop(0, png, _rz, 0)

            @pl.when(pnch > 1)
            def _():
                _zero_full(winb)

        _wait_inputs(kcb, vcb)
        ng0 = jnp.minimum((hi - lo_al + np.int32(15)) // np.int32(16),
                          np.int32(GR))

        def _group0(k, _):
            off = k * 16
            ka = kcb[pl.ds(off, 16)]
            kb = kcb[pl.ds(off + 1, 16)]
            va = vcb[pl.ds(off, 16)]
            gidx = lo_al + off + lane
            keep = (ka != kb) & (gidx >= lo) & (gidx < hi)
            outk = jnp.where(keep, ka - wsplat, lane * 0)
            outv = jnp.where(keep, va, tvsplat)
            plsc.store_scatter(winb, [outk], outv)
            sib[pl.ds(off, 16)] = outk
            return 0
        lax.fori_loop(0, ng0, _group0, 0)

        @pl.when(nch > 1)
        def _():
            def _chunk(c, _):
                base = pl.multiple_of(lo_al + c * np.int32(CH), 8)
                pltpu.sync_copy(skey_hbm.at[pl.ds(base, KCH)], kcb)
                pltpu.sync_copy(sval_hbm.at[pl.ds(base, CH)], vcb)

                def _group(k, _):
                    off = k * 16
                    ka = kcb[pl.ds(off, 16)]
                    kb = kcb[pl.ds(off + 1, 16)]
                    va = vcb[pl.ds(off, 16)]
                    gidx = base + off + lane
                    keep = (ka != kb) & (gidx >= lo) & (gidx < hi)
                    outk = jnp.where(keep, ka - wsplat, lane * 0)
                    outv = jnp.where(keep, va, tvsplat)
                    plsc.store_scatter(winb, [outk], outv)
                    return 0
                lax.fori_loop(0, GR, _group, 0)
                return 0
            lax.fori_loop(1, nch, _chunk, 0)

        _fire_inputs(p + np.int32(2), kcb, vcb)

        woff = pl.multiple_of(wstart, 8)

        @pl.when(r == NREG - 1)
        def _():
            pltpu.async_copy(
                winb.at[pl.ds(0, WLAST)], out_hbm.at[pl.ds(woff, WLAST)],
                osem)

        @pl.when(r != NREG - 1)
        def _():
            pltpu.async_copy(winb, out_hbm.at[pl.ds(woff, W)], osem)

    def _pair(pp, _):
        for b in range(2):
            winb, kcb, vcb, sib = bufs[b]
            _process(pp * 2 + np.int32(b), winb, kcb, vcb, sib)
        return 0
    lax.fori_loop(0, P // 2, _pair, 0)

    _wait_inputs(kc0, vc0)
    _wait_inputs(kc1, vc1)
    _wait_out_w(win0)

    @pl.when(r0 + np.int32(P - 1) == np.int32(NREG - 1))
    def _():
        _wait_out_last(win1)

    @pl.when(r0 + np.int32(P - 1) != np.int32(NREG - 1))
    def _():
        _wait_out_w(win1)


@jax.jit
def _sc_scatter(skey_pad, sval_pad, starts, tvals):
    mesh = plsc.VectorSubcoreMesh(core_axis_name="c", subcore_axis_name="s")
    f = functools.partial(
        pl.kernel,
        mesh=mesh,
        compiler_params=pltpu.CompilerParams(needs_layout_passes=False),
        out_type=jax.ShapeDtypeStruct((N * M,), jnp.float32),
        scratch_types=[
            pltpu.VMEM((W,), jnp.float32),
            pltpu.VMEM((W,), jnp.float32),
            pltpu.VMEM((KCH,), jnp.int32),
            pltpu.VMEM((KCH,), jnp.int32),
            pltpu.VMEM((CH,), jnp.float32),
            pltpu.VMEM((CH,), jnp.float32),
            pltpu.VMEM((CH,), jnp.int32),
            pltpu.VMEM((CH,), jnp.int32),
            pltpu.VMEM((SB_LEN,), jnp.int32),
            pltpu.VMEM((SB_LEN,), jnp.float32),
            pltpu.SemaphoreType.DMA,
            pltpu.SemaphoreType.DMA,
        ],
    )(_sc_body)
    return f(skey_pad, sval_pad, starts, tvals)


def kernel(node_features, data, indices):
    flat = indices[:, 0] * np.int32(M) + indices[:, 1]
    skey, sval = lax.sort_key_val(flat, data, is_stable=False)

    skey_pad = jnp.full((PAD_LEN,), SENTINEL, jnp.int32).at[:NNZ].set(skey)
    sval_pad = jnp.zeros((PAD_LEN,), jnp.float32).at[:NNZ].set(sval)

    targets = jnp.arange(NREG + 1, dtype=jnp.int32) * np.int32(W)
    bounds = jnp.searchsorted(skey, targets, side="left").astype(jnp.int32)
    starts = jnp.full((SB_LEN,), np.int32(NNZ), jnp.int32).at[:NREG + 1].set(
        bounds)

    rt = targets[:NREG]
    pr = jnp.searchsorted(skey, rt, side="right").astype(jnp.int32) - 1
    prc = jnp.maximum(pr, 0)
    exists = (pr >= 0) & (skey[prc] == rt)
    tvals = jnp.zeros((SB_LEN,), jnp.float32).at[:NREG].set(
        jnp.where(exists, sval[prc], 0.0))

    out = _sc_scatter(skey_pad, sval_pad, starts, tvals)
    return out.reshape(N, M)


# speedup vs baseline: 3.5404x; 1.0663x over previous
import functools

import numpy as np

import jax
import jax.numpy as jnp
from jax import lax
from jax.experimental import pallas as pl
from jax.experimental.pallas import tpu as pltpu
from jax.experimental.pallas import tpu_sc as plsc

N = 4096
M = 4096
NNZ = 167772
NW = 32
P = 10
NREG = NW * P
W = 52432
WLAST = N * M - (NREG - 1) * W
ZI = W // 16

CH = 2048
GR = CH // 16
KCH = CH + 32
PAD_LEN = ((NNZ + 2 * CH + KCH) // 8 + 1) * 8
SB_LEN = 352

SENTINEL = np.int32(0x7FFFFFFF)


def _sc_body(skey_hbm, sval_hbm, starts_hbm, tvals_hbm, out_hbm,
             win0, win1, kc0, kc1, vc0, vc1, si0, si1, sb, tb,
             isem, osem):
    wid = lax.axis_index("s") * 2 + lax.axis_index("c")
    r0 = wid * np.int32(P)

    pltpu.sync_copy(starts_hbm, sb)
    pltpu.sync_copy(tvals_hbm, tb)

    lane = lax.iota(jnp.int32, 16)
    zf16 = (lane * 0).astype(jnp.float32)

    def _region_info(p):
        bv = sb[pl.ds(r0 + p, 16)]
        lo = bv[0]
        hi = bv[1]
        lo_al = lo & np.int32(-8)
        return lo, hi, lo_al

    def _fire_inputs(p, kcb, vcb):
        _, _, lo_al = _region_info(p)
        abase = pl.multiple_of(lo_al, 8)
        pltpu.async_copy(skey_hbm.at[pl.ds(abase, KCH)], kcb, isem)
        pltpu.async_copy(sval_hbm.at[pl.ds(abase, CH)], vcb, isem)

    def _wait_inputs(kcb, vcb):
        pltpu.make_async_copy(
            skey_hbm.at[pl.ds(0, KCH)], kcb, isem).wait()
        pltpu.make_async_copy(
            sval_hbm.at[pl.ds(0, CH)], vcb, isem).wait()

    def _wait_out_w(winb):
        pltpu.make_async_copy(
            winb, out_hbm.at[pl.ds(0, W)], osem).wait()

    def _wait_out_last(winb):
        pltpu.make_async_copy(
            winb.at[pl.ds(0, WLAST)], out_hbm.at[pl.ds(0, WLAST)],
            osem).wait()

    def _zero_full(winb):
        def _z(i, _):
            for u in range(8):
                winb[pl.ds(i * 128 + u * 16, 16)] = zf16
            return 0
        lax.fori_loop(0, ZI // 8, _z, 0)
        for u in range(ZI % 8):
            winb[pl.ds((ZI // 8) * 128 + u * 16, 16)] = zf16

    _zero_full(win0)
    _zero_full(win1)
    _fire_inputs(np.int32(0), kc0, vc0)
    _fire_inputs(np.int32(1), kc1, vc1)

    bufs = ((win0, kc0, vc0, si0), (win1, kc1, vc1, si1))

    def _process(p, winb, kcb, vcb, sib):
        r = r0 + p
        lo, hi, lo_al = _region_info(p)
        nch = (hi - lo_al + np.int32(CH - 1)) // np.int32(CH)
        wstart = r * np.int32(W)
        wsplat = jnp.full((16,), wstart, jnp.int32)
        tval = tb[pl.ds(r, 16)][0]
        tvsplat = jnp.full((16,), tval, jnp.float32)

        @pl.when(p >= 2)
        def _():
            _wait_out_w(winb)
            plo, phi, plo_al = _region_info(p - 2)
            pnch = (phi - plo_al + np.int32(CH - 1)) // np.int32(CH)

            @pl.when(pnch <= 1)
            def _():
                png = (phi - plo_al + np.int32(15)) // np.int32(16)

                def _rz(k, _):
                    idxv = sib[pl.ds(k * 16, 16)]
                    plsc.store_scatter(winb, [idxv], zf16)
                    return 0
                lax.fori_loop(0, png, _rz, 0)

            @pl.when(pnch > 1)
            def _():
                _zero_full(winb)

        _wait_inputs(kcb, vcb)
        ng0 = jnp.minimum((hi - lo_al + np.int32(15)) // np.int32(16),
                          np.int32(GR))

        def _group0(k, _):
            off = k * 16
            ka = kcb[pl.ds(off, 16)]
            kb = kcb[pl.ds(off + 1, 16)]
            va = vcb[pl.ds(off, 16)]
            gidx = lo_al + off + lane
            keep = (ka != kb) & (gidx >= lo) & (gidx < hi)
            outk = jnp.where(keep, ka - wsplat, lane * 0)
            outv = jnp.where(keep, va, tvsplat)
            plsc.store_scatter(winb, [outk], outv)
            sib[pl.ds(off, 16)] = outk
            return 0
        lax.fori_loop(0, ng0, _group0, 0)

        @pl.when(nch > 1)
        def _():
            def _chunk(c, _):
                base = pl.multiple_of(lo_al + c * np.int32(CH), 8)
                pltpu.sync_copy(skey_hbm.at[pl.ds(base, KCH)], kcb)
                pltpu.sync_copy(sval_hbm.at[pl.ds(base, CH)], vcb)

                def _group(k, _):
                    off = k * 16
                    ka = kcb[pl.ds(off, 16)]
                    kb = kcb[pl.ds(off + 1, 16)]
                    va = vcb[pl.ds(off, 16)]
                    gidx = base + off + lane
                    keep = (ka != kb) & (gidx >= lo) & (gidx < hi)
                    outk = jnp.where(keep, ka - wsplat, lane * 0)
                    outv = jnp.where(keep, va, tvsplat)
                    plsc.store_scatter(winb, [outk], outv)
                    return 0
                lax.fori_loop(0, GR, _group, 0)
                return 0
            lax.fori_loop(1, nch, _chunk, 0)

        _fire_inputs(p + np.int32(2), kcb, vcb)

        woff = pl.multiple_of(wstart, 8)

        @pl.when(r == NREG - 1)
        def _():
            pltpu.async_copy(
                winb.at[pl.ds(0, WLAST)], out_hbm.at[pl.ds(woff, WLAST)],
                osem)

        @pl.when(r != NREG - 1)
        def _():
            pltpu.async_copy(winb, out_hbm.at[pl.ds(woff, W)], osem)

    def _pair(pp, _):
        for b in range(2):
            winb, kcb, vcb, sib = bufs[b]
            _process(pp * 2 + np.int32(b), winb, kcb, vcb, sib)
        return 0
    lax.fori_loop(0, P // 2, _pair, 0)

    _wait_inputs(kc0, vc0)
    _wait_inputs(kc1, vc1)
    _wait_out_w(win0)

    @pl.when(r0 + np.int32(P - 1) == np.int32(NREG - 1))
    def _():
        _wait_out_last(win1)

    @pl.when(r0 + np.int32(P - 1) != np.int32(NREG - 1))
    def _():
        _wait_out_w(win1)


@jax.jit
def _sc_scatter(skey_pad, sval_pad, starts, tvals):
    mesh = plsc.VectorSubcoreMesh(core_axis_name="c", subcore_axis_name="s")
    f = functools.partial(
        pl.kernel,
        mesh=mesh,
        compiler_params=pltpu.CompilerParams(needs_layout_passes=False),
        out_type=jax.ShapeDtypeStruct((N * M,), jnp.float32),
        scratch_types=[
            pltpu.VMEM((W,), jnp.float32),
            pltpu.VMEM((W,), jnp.float32),
            pltpu.VMEM((KCH,), jnp.int32),
            pltpu.VMEM((KCH,), jnp.int32),
            pltpu.VMEM((CH,), jnp.float32),
            pltpu.VMEM((CH,), jnp.float32),
            pltpu.VMEM((CH,), jnp.int32),
            pltpu.VMEM((CH,), jnp.int32),
            pltpu.VMEM((SB_LEN,), jnp.int32),
            pltpu.VMEM((SB_LEN,), jnp.float32),
            pltpu.SemaphoreType.DMA,
            pltpu.SemaphoreType.DMA,
        ],
    )(_sc_body)
    return f(skey_pad, sval_pad, starts, tvals)


def kernel(node_features, data, indices):
    flat = indices[:, 0] * np.int32(M) + indices[:, 1]
    skey, sval = lax.sort_key_val(flat, data, is_stable=False)

    skey_pad = jnp.full((PAD_LEN,), SENTINEL, jnp.int32).at[:NNZ].set(skey)
    sval_pad = jnp.zeros((PAD_LEN,), jnp.float32).at[:NNZ].set(sval)

    targets = jnp.arange(NREG + 1, dtype=jnp.int32) * np.int32(W)
    bounds = jnp.searchsorted(skey, targets, side="left").astype(jnp.int32)
    starts = jnp.full((SB_LEN,), np.int32(NNZ), jnp.int32).at[:NREG + 1].set(
        bounds)

    rt = targets[:NREG]
    pr = jnp.searchsorted(skey, rt, side="right").astype(jnp.int32) - 1
    prc = jnp.maximum(pr, 0)
    exists = (pr >= 0) & (skey[prc] == rt)
    tvals = jnp.zeros((SB_LEN,), jnp.float32).at[:NREG].set(
        jnp.where(exists, sval[prc], 0.0))

    out = _sc_scatter(skey_pad, sval_pad, starts, tvals)
    return out.reshape(N, M)
